# Initial kernel scaffold; baseline (speedup 1.0000x reference)
import functools

import jax
import jax.numpy as jnp
from jax import lax
from jax.experimental import pallas as pl
from jax.experimental.pallas import tpu as pltpu
from jax.experimental.pallas import tpu_sc as plsc


def _make_pool(B, Lc, Ld, D):
    info = plsc.get_sparse_core_info()
    NC, NS = info.num_cores, info.num_subcores
    NW = NC * NS
    RW = B // NW
    CBC = 8
    CBD = 2
    NIDX = CBC * Lc
    assert CBD * Ld == NIDX
    NK = D // 16

    mesh = plsc.VectorSubcoreMesh(core_axis_name="c", subcore_axis_name="s")

    @functools.partial(
        pl.kernel,
        out_type=(jax.ShapeDtypeStruct((B, D), jnp.float32),
                  jax.ShapeDtypeStruct((B, D), jnp.float32)),
        mesh=mesh,
        scratch_types=[
            pltpu.VMEM((NIDX,), jnp.int32),
            pltpu.VMEM((NIDX, D), jnp.float32),
            pltpu.VMEM((CBC, D), jnp.float32),
            pltpu.SemaphoreType.DMA,
        ],
    )
    def pool(ctx_hbm, doc_hbm, table_hbm, octx_hbm, odoc_hbm,
             idx_v, rows_v, out_v, sem):
        wid = lax.axis_index("s") * NC + lax.axis_index("c")
        row0 = wid * RW

        def one_pass(idx_hbm, out_hbm, L, CB):
            scale = jnp.float32(1.0 / L)

            def chunk(c, carry):
                r0 = row0 + c * CB
                pltpu.sync_copy(idx_hbm.at[pl.ds(r0 * L, CB * L)],
                                idx_v.at[pl.ds(0, CB * L)])
                pltpu.async_copy(table_hbm.at[idx_v], rows_v, sem).wait()
                for r in range(CB):
                    def jbody(j, accs, r=r):
                        jj = r * L + j
                        return tuple(accs[k] + rows_v[jj, pl.ds(16 * k, 16)]
                                     for k in range(NK))
                    accs = lax.fori_loop(
                        0, L, jbody,
                        tuple(jnp.zeros((16,), jnp.float32)
                              for _ in range(NK)))
                    for k in range(NK):
                        out_v[r, pl.ds(16 * k, 16)] = accs[k] * scale
                pltpu.sync_copy(out_v.at[pl.ds(0, CB)],
                                out_hbm.at[pl.ds(r0, CB)])
                return carry

            lax.fori_loop(0, RW // CB, chunk, 0)

        one_pass(ctx_hbm, octx_hbm, Lc, CBC)
        one_pass(doc_hbm, odoc_hbm, Ld, CBD)

    return pool


def _mlp_body(c_ref, d_ref, w1_ref, w2_ref, b_ref, o_ref):
    o_ref[...] = (
        jnp.dot(c_ref[...], w1_ref[...], preferred_element_type=jnp.float32)
        + jnp.dot(d_ref[...], w2_ref[...], preferred_element_type=jnp.float32)
        + b_ref[...])


def _mlp(ctx_vec, doc_vec, W1, W2, b2d):
    B, D = ctx_vec.shape
    BB = 512
    return pl.pallas_call(
        _mlp_body,
        out_shape=jax.ShapeDtypeStruct((B, D), jnp.float32),
        grid=(B // BB,),
        in_specs=[
            pl.BlockSpec((BB, D), lambda i: (i, 0)),
            pl.BlockSpec((BB, D), lambda i: (i, 0)),
            pl.BlockSpec((D, D), lambda i: (0, 0)),
            pl.BlockSpec((D, D), lambda i: (0, 0)),
            pl.BlockSpec((1, D), lambda i: (0, 0)),
        ],
        out_specs=pl.BlockSpec((BB, D), lambda i: (i, 0)),
    )(ctx_vec, doc_vec, W1, W2, b2d)


def kernel(context, doc, word_embeds, W_mlp, b_mlp):
    B, Lc = context.shape
    _, Ld = doc.shape
    _, D = word_embeds.shape
    ctx_flat = context.reshape(B * Lc)
    doc_flat = doc.reshape(B * Ld)
    pool = _make_pool(B, Lc, Ld, D)
    ctx_vec, doc_vec = pool(ctx_flat, doc_flat, word_embeds)
    return _mlp(ctx_vec, doc_vec, W_mlp[:D], W_mlp[D:], b_mlp.reshape(1, D))


# speedup vs baseline: 10.0883x; 10.0883x over previous
import functools

import jax
import jax.numpy as jnp
from jax import lax
from jax.experimental import pallas as pl
from jax.experimental.pallas import tpu as pltpu
from jax.experimental.pallas import tpu_sc as plsc


def _make_pool(B, Lc, Ld, D):
    info = plsc.get_sparse_core_info()
    NC, NS = info.num_cores, info.num_subcores
    NW = NC * NS
    RW = B // NW
    CBC = 8
    CBD = 2
    NIDX = CBC * Lc
    assert CBD * Ld == NIDX
    NK = D // 16

    mesh = plsc.VectorSubcoreMesh(core_axis_name="c", subcore_axis_name="s")

    @functools.partial(
        pl.kernel,
        out_type=(jax.ShapeDtypeStruct((B, D), jnp.float32),
                  jax.ShapeDtypeStruct((B, D), jnp.float32)),
        mesh=mesh,
        scratch_types=[
            pltpu.VMEM((NIDX,), jnp.int32),
            pltpu.VMEM((NIDX, D), jnp.float32),
            pltpu.VMEM((CBC, D), jnp.float32),
            pltpu.SemaphoreType.DMA,
        ],
        compiler_params=pltpu.CompilerParams(use_tc_tiling_on_sc=False),
    )
    def pool(ctx_hbm, doc_hbm, table_hbm, octx_hbm, odoc_hbm,
             idx_v, rows_v, out_v, sem):
        wid = lax.axis_index("s") * NC + lax.axis_index("c")
        row0 = wid * RW

        def one_pass(idx_hbm, out_hbm, L, CB):
            scale = jnp.float32(1.0 / L)

            def chunk(c, carry):
                r0 = row0 + c * CB
                pltpu.sync_copy(idx_hbm.at[pl.ds(r0 * L, CB * L)],
                                idx_v.at[pl.ds(0, CB * L)])
                pltpu.async_copy(table_hbm.at[idx_v], rows_v, sem).wait()
                for r in range(CB):
                    def jbody(j, accs, r=r):
                        jj = r * L + j
                        return tuple(accs[k] + rows_v[jj, pl.ds(16 * k, 16)]
                                     for k in range(NK))
                    accs = lax.fori_loop(
                        0, L, jbody,
                        tuple(jnp.zeros((16,), jnp.float32)
                              for _ in range(NK)))
                    for k in range(NK):
                        out_v[r, pl.ds(16 * k, 16)] = accs[k] * scale
                pltpu.sync_copy(out_v.at[pl.ds(0, CB)],
                                out_hbm.at[pl.ds(r0, CB)])
                return carry

            lax.fori_loop(0, RW // CB, chunk, 0)

        one_pass(ctx_hbm, octx_hbm, Lc, CBC)
        one_pass(doc_hbm, odoc_hbm, Ld, CBD)

    return pool


def _mlp_body(c_ref, d_ref, w1_ref, w2_ref, b_ref, o_ref):
    o_ref[...] = (
        jnp.dot(c_ref[...], w1_ref[...], preferred_element_type=jnp.float32)
        + jnp.dot(d_ref[...], w2_ref[...], preferred_element_type=jnp.float32)
        + b_ref[...])


def _mlp(ctx_vec, doc_vec, W1, W2, b2d):
    B, D = ctx_vec.shape
    BB = 512
    return pl.pallas_call(
        _mlp_body,
        out_shape=jax.ShapeDtypeStruct((B, D), jnp.float32),
        grid=(B // BB,),
        in_specs=[
            pl.BlockSpec((BB, D), lambda i: (i, 0)),
            pl.BlockSpec((BB, D), lambda i: (i, 0)),
            pl.BlockSpec((D, D), lambda i: (0, 0)),
            pl.BlockSpec((D, D), lambda i: (0, 0)),
            pl.BlockSpec((1, D), lambda i: (0, 0)),
        ],
        out_specs=pl.BlockSpec((BB, D), lambda i: (i, 0)),
    )(ctx_vec, doc_vec, W1, W2, b2d)


def kernel(context, doc, word_embeds, W_mlp, b_mlp):
    B, Lc = context.shape
    _, Ld = doc.shape
    _, D = word_embeds.shape
    ctx_flat = context.reshape(B * Lc)
    doc_flat = doc.reshape(B * Ld)
    pool = _make_pool(B, Lc, Ld, D)
    ctx_vec, doc_vec = pool(ctx_flat, doc_flat, word_embeds)
    return _mlp(ctx_vec, doc_vec, W_mlp[:D], W_mlp[D:], b_mlp.reshape(1, D))


# speedup vs baseline: 18.1801x; 1.8021x over previous
import functools

import jax
import jax.numpy as jnp
from jax import lax
from jax.experimental import pallas as pl
from jax.experimental.pallas import tpu as pltpu
from jax.experimental.pallas import tpu_sc as plsc


def _make_pool(B, Lc, Ld, D):
    info = plsc.get_sparse_core_info()
    NC, NS = info.num_cores, info.num_subcores
    NW = NC * NS
    RW = B // NW
    CBC = 8
    CBD = 2
    NIDX = CBC * Lc
    assert CBD * Ld == NIDX
    NK = D // 16

    mesh = plsc.VectorSubcoreMesh(core_axis_name="c", subcore_axis_name="s")

    @functools.partial(
        pl.kernel,
        out_type=(jax.ShapeDtypeStruct((B, D), jnp.float32),
                  jax.ShapeDtypeStruct((B, D), jnp.float32)),
        mesh=mesh,
        scratch_types=[
            pltpu.VMEM((RW * Lc,), jnp.int32),
            pltpu.VMEM((RW * Ld,), jnp.int32),
            pltpu.VMEM((NIDX, D), jnp.float32),
            pltpu.VMEM((NIDX, D), jnp.float32),
            pltpu.VMEM((CBC, D), jnp.float32),
            pltpu.SemaphoreType.DMA,
            pltpu.SemaphoreType.DMA,
            pltpu.SemaphoreType.DMA,
        ],
        compiler_params=pltpu.CompilerParams(use_tc_tiling_on_sc=False),
    )
    def pool(ctx_hbm, doc_hbm, table_hbm, octx_hbm, odoc_hbm,
             idxc_v, idxd_v, rows0_v, rows1_v, out_v, sem0, sem1, semi):
        wid = lax.axis_index("s") * NC + lax.axis_index("c")
        row0 = wid * RW
        rows = (rows0_v, rows1_v)
        sems = (sem0, sem1)

        pltpu.sync_copy(ctx_hbm.at[pl.ds(row0 * Lc, RW * Lc)], idxc_v)
        pltpu.async_copy(doc_hbm.at[pl.ds(row0 * Ld, RW * Ld)], idxd_v,
                         semi)

        def one_pass(idx_v, out_hbm, L, CB):
            scale = jnp.float32(1.0 / L)
            nch = RW // CB

            def start(c, b):
                pltpu.async_copy(
                    table_hbm.at[idx_v.at[pl.ds(c * CB * L, CB * L)]],
                    rows[b], sems[b])

            def wait(b):
                pltpu.make_async_copy(
                    table_hbm.at[pl.ds(0, NIDX)], rows[b], sems[b]).wait()

            def accum(c, b):
                rbuf = rows[b]
                for r in range(CB):
                    def jbody(j, accs, r=r):
                        jj = (r * L + 2 * j)
                        out = []
                        for k in range(NK):
                            a = accs[k] + rbuf[jj, pl.ds(16 * k, 16)]
                            out.append(a + rbuf[jj + 1, pl.ds(16 * k, 16)])
                        return tuple(out)
                    accs = lax.fori_loop(
                        0, L // 2, jbody,
                        tuple(jnp.zeros((16,), jnp.float32)
                              for _ in range(NK)))
                    for k in range(NK):
                        out_v[r, pl.ds(16 * k, 16)] = accs[k] * scale
                pltpu.sync_copy(out_v.at[pl.ds(0, CB)],
                                out_hbm.at[pl.ds(row0 + c * CB, CB)])

            start(0, 0)

            def pair(p, carry):
                c0 = 2 * p
                start(c0 + 1, 1)
                wait(0)
                accum(c0, 0)

                @pl.when(p + 1 < nch // 2)
                def _():
                    start(c0 + 2, 0)

                wait(1)
                accum(c0 + 1, 1)
                return carry

            lax.fori_loop(0, nch // 2, pair, 0)

        one_pass(idxc_v, octx_hbm, Lc, CBC)
        pltpu.make_async_copy(
            doc_hbm.at[pl.ds(0, RW * Ld)], idxd_v, semi).wait()
        one_pass(idxd_v, odoc_hbm, Ld, CBD)

    return pool


def _mlp_body(c_ref, d_ref, w1_ref, w2_ref, b_ref, o_ref):
    o_ref[...] = (
        jnp.dot(c_ref[...], w1_ref[...], preferred_element_type=jnp.float32)
        + jnp.dot(d_ref[...], w2_ref[...], preferred_element_type=jnp.float32)
        + b_ref[...])


def _mlp(ctx_vec, doc_vec, W1, W2, b2d):
    B, D = ctx_vec.shape
    BB = 512
    return pl.pallas_call(
        _mlp_body,
        out_shape=jax.ShapeDtypeStruct((B, D), jnp.float32),
        grid=(B // BB,),
        in_specs=[
            pl.BlockSpec((BB, D), lambda i: (i, 0)),
            pl.BlockSpec((BB, D), lambda i: (i, 0)),
            pl.BlockSpec((D, D), lambda i: (0, 0)),
            pl.BlockSpec((D, D), lambda i: (0, 0)),
            pl.BlockSpec((1, D), lambda i: (0, 0)),
        ],
        out_specs=pl.BlockSpec((BB, D), lambda i: (i, 0)),
    )(ctx_vec, doc_vec, W1, W2, b2d)


def kernel(context, doc, word_embeds, W_mlp, b_mlp):
    B, Lc = context.shape
    _, Ld = doc.shape
    _, D = word_embeds.shape
    ctx_flat = context.reshape(B * Lc)
    doc_flat = doc.reshape(B * Ld)
    pool = _make_pool(B, Lc, Ld, D)
    ctx_vec, doc_vec = pool(ctx_flat, doc_flat, word_embeds)
    return _mlp(ctx_vec, doc_vec, W_mlp[:D], W_mlp[D:], b_mlp.reshape(1, D))
